# baseline (speedup 1.0000x reference)
import functools

import jax
import jax.numpy as jnp
from jax import lax
from jax.experimental import pallas as pl
from jax.experimental.pallas import tpu as pltpu
from jax.experimental.pallas import tpu_sc as plsc

_N_NODES = 10000
_D = 128
_E = 320000
_NC = 2
_NS = 16
_NW = _NC * _NS
_EPW = _E // _NW
_C = 16
_NCHUNK = _EPW // _C
_G = _C // 16


def _dot_chunk(sb, db, outv, dots, off, last_lane):
    for e in range(_C):
        p = sb[e, pl.ds(0, 16)] * db[e, pl.ds(0, 16)]
        for c in range(1, _D // 16):
            p = p + sb[e, pl.ds(c * 16, 16)] * db[e, pl.ds(c * 16, 16)]
        cum = plsc.cumsum(p)
        plsc.store_scatter(dots, [jnp.full((16,), e, jnp.int32)], cum,
                           mask=last_lane)
    v = dots[...]
    outv[pl.ds(off, 16)] = 1.0 / (1.0 + jnp.exp(-v))


_NSLOT = 4


def _edge_kernel(x_hbm, edge_hbm, out_hbm, x_s, sidx, didx,
                 sb0, db0, sb1, db1, sb2, db2, sb3, db3, outv, dots,
                 ss0, sd0, ss1, sd1, ss2, sd2, ss3, sd3):
    cid = lax.axis_index("c")
    sid = lax.axis_index("s")
    wid = sid * _NC + cid
    base = wid * _EPW

    rows_per = 640
    @pl.when(sid < _NS - 1)
    def _():
        pltpu.sync_copy(x_hbm.at[pl.ds(sid * rows_per, rows_per)],
                        x_s.at[pl.ds(sid * rows_per, rows_per)])
    @pl.when(sid == _NS - 1)
    def _():
        last = (_NS - 1) * rows_per
        pltpu.sync_copy(x_hbm.at[pl.ds(last, _N_NODES - last)],
                        x_s.at[pl.ds(last, _N_NODES - last)])
    pltpu.sync_copy(edge_hbm.at[pl.ds(base, _EPW)], sidx)
    pltpu.sync_copy(edge_hbm.at[pl.ds(_E + base, _EPW)], didx)
    plsc.subcore_barrier()

    last_lane = lax.broadcasted_iota(jnp.int32, (16,), 0) == 15

    def start(cc, sb, db, ss, sd):
        o = cc * _C
        pltpu.async_copy(x_s.at[sidx.at[pl.ds(o, _C)]], sb, ss)
        pltpu.async_copy(x_s.at[didx.at[pl.ds(o, _C)]], db, sd)

    def wait(sb, db, ss, sd):
        pltpu.make_async_copy(x_s.at[pl.ds(0, _C)], sb, ss).wait()
        pltpu.make_async_copy(x_s.at[pl.ds(0, _C)], db, sd).wait()

    slots = ((sb0, db0, ss0, sd0), (sb1, db1, ss1, sd1),
             (sb2, db2, ss2, sd2), (sb3, db3, ss3, sd3))
    for par in range(_NSLOT):
        start(par, *slots[par])

    @pl.loop(0, _NCHUNK - 1, step=_NSLOT)
    def _round(c):
        for par in range(_NSLOT):
            sb, db, ss, sd = slots[par]
            cc = c + par
            wait(sb, db, ss, sd)
            _dot_chunk(sb, db, outv, dots, cc * _C, last_lane)

            @pl.when(cc + _NSLOT < _NCHUNK)
            def _(cc=cc, sb=sb, db=db, ss=ss, sd=sd):
                start(cc + _NSLOT, sb, db, ss, sd)

    wait(*slots[0])
    _dot_chunk(sb0, db0, outv, dots, (_NCHUNK - 1) * _C, last_lane)

    pltpu.sync_copy(outv, out_hbm.at[pl.ds(base, _EPW)])


@jax.jit
def kernel(x, edge_index):
    mesh = plsc.VectorSubcoreMesh(core_axis_name="c", subcore_axis_name="s",
                                  num_cores=_NC, num_subcores=_NS)
    f = pl.kernel(
        _edge_kernel,
        out_type=jax.ShapeDtypeStruct((_E,), jnp.float32),
        mesh=mesh,
        compiler_params=pltpu.CompilerParams(needs_layout_passes=False),
        scratch_types=[
            pltpu.VMEM_SHARED((_N_NODES, _D), jnp.float32),
            pltpu.VMEM((_EPW,), jnp.int32),
            pltpu.VMEM((_EPW,), jnp.int32),
        ] + [
            pltpu.VMEM((_C, _D), jnp.float32)
            for _ in range(2 * _NSLOT)
        ] + [
            pltpu.VMEM((_EPW,), jnp.float32),
            pltpu.VMEM((16,), jnp.float32),
        ] + [pltpu.SemaphoreType.DMA for _ in range(2 * _NSLOT)],
    )
    return f(x, edge_index.reshape(2 * _E))
